# baseline (speedup 1.0000x reference)
import functools

import jax
import jax.numpy as jnp
from jax import lax
from jax.experimental import pallas as pl
from jax.experimental.pallas import tpu as pltpu
from jax.experimental.pallas import tpu_sc as plsc

_M = 8
_P = 4

_NC = 2
_NS = 16
_NW = _NC * _NS
_CH = 128


def _prep_argmin_body(q_ref, qp_ref, wall_ref, ball_ref, inp_ref, wval_ref,
                      bval_ref, ip4_ref, ip2_ref, ones4_ref,
                      attw_ref, val_ref, out_ref, samp_s, sp2_s, *, Lq, Lin):
    G = _M * _P
    b = pl.program_id(0)
    g = pl.program_id(1)

    @pl.when(g == 0)
    def _prep():
        proj = jnp.dot(q_ref[0], wall_ref[...],
                       preferred_element_type=jnp.float32) + ball_ref[...]
        qp_z = jnp.concatenate(
            [qp_ref[0], jnp.zeros((Lq, 5), jnp.float32)], axis=1)
        qp_t = jnp.concatenate([qp_z] * G, axis=1)
        samp8 = qp_t + proj[:, :8 * G]
        samp_s[...] = samp8.T
        sp2_s[...] = jnp.dot(samp8 * samp8, ones4_ref[...],
                             preferred_element_type=jnp.float32,
                             precision=lax.Precision.HIGHEST)
        logits = proj[:, 8 * G:9 * G]
        sl = [logits[:, i * _M:(i + 1) * _M] for i in range(_P)]
        mx = jnp.maximum(jnp.maximum(sl[0], sl[1]),
                         jnp.maximum(sl[2], sl[3]))
        e = jnp.exp(logits - jnp.concatenate([mx] * _P, axis=1))
        es = [e[:, i * _M:(i + 1) * _M] for i in range(_P)]
        ssum = (es[0] + es[1]) + (es[2] + es[3])
        attw_ref[0] = e / jnp.concatenate([ssum] * _P, axis=1)
        value = jnp.dot(inp_ref[0], wval_ref[...],
                        preferred_element_type=jnp.float32) + bval_ref[...]
        val_ref[0, 0] = value[:, :128]
        val_ref[0, 1] = value[:, 128:]

    spg = samp_s[pl.ds(g * 8, 8), :]
    onehot = (lax.broadcasted_iota(jnp.int32, (G, 1), 0) == g
              ).astype(jnp.float32)
    sp2c = jnp.dot(sp2_s[...], onehot,
                   preferred_element_type=jnp.float32,
                   precision=lax.Precision.HIGHEST)
    d2 = (sp2c + ip2_ref[0]) - 2.0 * lax.dot_general(
        spg, ip4_ref[0], (((0,), (0,)), ((), ())),
        preferred_element_type=jnp.float32)
    mn = jnp.min(d2, axis=1, keepdims=True)
    lane = lax.broadcasted_iota(jnp.int32, d2.shape, 1)
    idx = jnp.min(jnp.where(d2 == mn, lane, Lin), axis=1, keepdims=True)
    h = g // 16
    out_ref[0, 0] = idx + (b * 2 + h) * Lin


def _combine_body(h_ref, w_ref, b_ref, o_ref):
    h = h_ref[0]
    h0 = h[:, 0, :]
    h1 = h[:, 1, :]
    o_ref[0] = (jnp.dot(h0, w_ref[0:128, :],
                        preferred_element_type=jnp.float32)
                + jnp.dot(h1, w_ref[128:256, :],
                          preferred_element_type=jnp.float32)
                + b_ref[...])


def _sc_gather_combine(table, idx3, attw3, n_out, d):
    rows_w = (n_out * _P) // _NW
    nch = rows_w // _CH
    out_rows_ch = _CH // _P // _P
    mesh = plsc.VectorSubcoreMesh(core_axis_name="c", subcore_axis_name="s")

    @functools.partial(
        pl.kernel,
        mesh=mesh,
        out_type=jax.ShapeDtypeStruct((n_out // _P, 4 * d), jnp.float32),
        scratch_types=[
            pltpu.VMEM((nch, _CH), jnp.int32),
            pltpu.VMEM((rows_w,), jnp.float32),
            pltpu.VMEM((_CH, 4 * d), jnp.float32),
            pltpu.VMEM((_CH, 4 * d), jnp.float32),
            pltpu.VMEM((out_rows_ch, 4 * d), jnp.float32),
            pltpu.SemaphoreType.DMA,
            pltpu.SemaphoreType.DMA,
        ],
        compiler_params=pltpu.CompilerParams(use_tc_tiling_on_sc=True,
                                             needs_layout_passes=False),
    )
    def gc_kernel(table_hbm, idx_hbm, attw_hbm, out_hbm,
                  idx_v, aw_v, raw_a, raw_b, out_v, sem_a, sem_b):
        wid = lax.axis_index("s") * _NC + lax.axis_index("c")
        pltpu.sync_copy(idx_hbm.at[wid], idx_v)
        pltpu.sync_copy(attw_hbm.at[wid], aw_v)

        def combine_chunk(c, raw_v):
            for j in range(_CH // _P):
                m = j % _M
                lb = (m % _P) * d
                pr = j // _P
                acc_lo = jnp.zeros((16,), jnp.float32)
                acc_hi = jnp.zeros((16,), jnp.float32)
                for p in range(_P):
                    n_loc = c * _CH + j * _P + p
                    w = plsc.load_gather(
                        aw_v, [jnp.full((16,), n_loc, jnp.int32)])
                    r = j * _P + p
                    acc_lo = acc_lo + w * raw_v[r, pl.ds(lb, 16)]
                    acc_hi = acc_hi + w * raw_v[r, pl.ds(lb + 16, 16)]
                out_v[pr, pl.ds(lb, 16)] = acc_lo
                out_v[pr, pl.ds(lb + 16, 16)] = acc_hi
            pltpu.sync_copy(
                out_v,
                out_hbm.at[pl.ds(wid * (nch * out_rows_ch) + c * out_rows_ch,
                                 out_rows_ch)])

        pltpu.async_copy(table_hbm.at[idx_v.at[0]], raw_a, sem_a)

        def pair(i, carry):
            ca = 2 * i
            pltpu.make_async_copy(
                table_hbm.at[idx_v.at[ca]], raw_a, sem_a).wait()
            cp_b = pltpu.async_copy(
                table_hbm.at[idx_v.at[ca + 1]], raw_b, sem_b)
            combine_chunk(ca, raw_a)
            cp_b.wait()

            @pl.when(i < nch // 2 - 1)
            def _():
                pltpu.async_copy(
                    table_hbm.at[idx_v.at[ca + 2]], raw_a, sem_a)

            combine_chunk(ca + 1, raw_b)
            return carry

        lax.fori_loop(0, nch // 2, pair, 0)

    return gc_kernel(table, idx3, attw3)


def kernel(query, query_points, input, input_points, W_off, b_off,
           W_attn, b_attn, W_val, b_val, W_out, b_out):
    B, Lq, C = query.shape
    Lin = input.shape[1]
    M, P = _M, _P
    G = M * P
    D = C // M
    N = Lq * G
    T = 512

    f32 = jnp.float32

    W_off8 = jnp.pad(W_off.reshape(C, G, 3),
                     ((0, 0), (0, 0), (0, 5))).reshape(C, 8 * G)
    b_off8 = jnp.pad(b_off.reshape(G, 3), ((0, 0), (0, 5))).reshape(8 * G)
    W_attn_r = W_attn.reshape(C, M, P).transpose(0, 2, 1).reshape(C, G)
    b_attn_r = b_attn.reshape(M, P).transpose(1, 0).reshape(G)
    W_all = jnp.concatenate([W_off8, W_attn_r], axis=1)
    b_all = jnp.concatenate([b_off8, b_attn_r])[None]
    ip8 = jnp.pad(input_points.transpose(0, 2, 1),
                  ((0, 0), (0, 5), (0, 0)))
    ip2 = jnp.sum(input_points * input_points, axis=-1)[:, None, :]
    ones8 = jnp.kron(jnp.eye(G, dtype=f32),
                     jnp.ones((8, 1), f32))
    b_out2 = b_out[None]

    attw_r, value4, gidx_g = pl.pallas_call(
        functools.partial(_prep_argmin_body, Lq=Lq, Lin=Lin),
        grid=(B, G),
        in_specs=[
            pl.BlockSpec((1, Lq, C), lambda b, g: (b, 0, 0)),
            pl.BlockSpec((1, Lq, 3), lambda b, g: (b, 0, 0)),
            pl.BlockSpec((C, 9 * G), lambda b, g: (0, 0)),
            pl.BlockSpec((1, 9 * G), lambda b, g: (0, 0)),
            pl.BlockSpec((1, Lin, C), lambda b, g: (b, 0, 0)),
            pl.BlockSpec((C, C), lambda b, g: (0, 0)),
            pl.BlockSpec((1, C), lambda b, g: (0, 0)),
            pl.BlockSpec((1, 8, Lin), lambda b, g: (b, 0, 0)),
            pl.BlockSpec((1, 1, Lin), lambda b, g: (b, 0, 0)),
            pl.BlockSpec((8 * G, G), lambda b, g: (0, 0)),
        ],
        out_specs=[
            pl.BlockSpec((1, Lq, G), lambda b, g: (b, 0, 0)),
            pl.BlockSpec((1, 2, Lin, 128), lambda b, g: (b, 0, 0, 0)),
            pl.BlockSpec((1, 1, Lq, 1), lambda b, g: (b, g, 0, 0)),
        ],
        out_shape=[
            jax.ShapeDtypeStruct((B, Lq, G), f32),
            jax.ShapeDtypeStruct((B, 2, Lin, 128), f32),
            jax.ShapeDtypeStruct((B, G, Lq, 1), jnp.int32),
        ],
        scratch_shapes=[
            pltpu.VMEM((8 * G, Lq), f32),
            pltpu.VMEM((Lq, G), f32),
        ],
    )(query, query_points, W_all, b_all, input, W_val, b_val[None],
      ip8, ip2, ones8)
    gidx = gidx_g.reshape(B, G, Lq).transpose(0, 2, 1)

    value_t = value4.reshape(B * 2 * Lin, 128)
    n_rows = B * N
    idx3 = gidx.reshape(_NW, n_rows // _NW // _CH, _CH)
    attw_m = attw_r.reshape(B, Lq, P, M).transpose(0, 1, 3, 2)
    attw3 = attw_m.reshape(_NW, n_rows // _NW)
    heads = _sc_gather_combine(value_t, idx3, attw3, B * Lq * M, D)

    h4 = heads.reshape(B, Lq, 2, 128)
    out = pl.pallas_call(
        _combine_body,
        grid=(B,),
        in_specs=[
            pl.BlockSpec((1, Lq, 2, 128), lambda b: (b, 0, 0, 0)),
            pl.BlockSpec((C, C), lambda b: (0, 0)),
            pl.BlockSpec((1, C), lambda b: (0, 0)),
        ],
        out_specs=pl.BlockSpec((1, Lq, C), lambda b: (b, 0, 0)),
        out_shape=jax.ShapeDtypeStruct((B, Lq, C), f32),
    )(h4, W_out, b_out2)
    return out


# speedup vs baseline: 1.0669x; 1.0669x over previous
import functools

import jax
import jax.numpy as jnp
from jax import lax
from jax.experimental import pallas as pl
from jax.experimental.pallas import tpu as pltpu
from jax.experimental.pallas import tpu_sc as plsc

_M = 8
_P = 4

_NC = 2
_NS = 16
_NW = _NC * _NS
_CH = 128


def _prep_body(q_ref, qp_ref, wall_ref, ball_ref, inp_ref, wval_ref,
               bval_ref, samp_ref, attw_ref, val_ref):
    G = _M * _P
    proj = jnp.dot(q_ref[0], wall_ref[...],
                   preferred_element_type=jnp.float32) + ball_ref[...]
    qp_t = jnp.concatenate([qp_ref[0]] * G, axis=1)
    samp_ref[0] = qp_t + proj[:, :3 * G]
    logits = proj[:, 3 * G:3 * G + G]
    sl = [logits[:, i * _M:(i + 1) * _M] for i in range(_P)]
    mx = jnp.maximum(jnp.maximum(sl[0], sl[1]), jnp.maximum(sl[2], sl[3]))
    e = jnp.exp(logits - jnp.concatenate([mx] * _P, axis=1))
    es = [e[:, i * _M:(i + 1) * _M] for i in range(_P)]
    ssum = (es[0] + es[1]) + (es[2] + es[3])
    attw_ref[0] = e / jnp.concatenate([ssum] * _P, axis=1)
    value = jnp.dot(inp_ref[0], wval_ref[...],
                    preferred_element_type=jnp.float32) + bval_ref[...]
    val_ref[0, 0] = value[:, :128]
    val_ref[0, 1] = value[:, 128:]


def _argmin_body(sp_ref, ipt_ref, ip2_ref, out_ref, *, T, Lin):
    b = pl.program_id(0)
    ipt = ipt_ref[0]
    ip2 = ip2_ref[0]

    def half(sp):
        sp2 = jnp.sum(sp * sp, axis=1, keepdims=True)
        d2 = (sp2 + ip2) - 2.0 * jnp.dot(
            sp, ipt, preferred_element_type=jnp.float32)
        mn = jnp.min(d2, axis=1, keepdims=True)
        lane = lax.broadcasted_iota(jnp.int32, d2.shape, 1)
        return jnp.min(jnp.where(d2 == mn, lane, Lin), axis=1, keepdims=True)

    idx = jnp.concatenate(
        [half(sp_ref[0, :T // 2]), half(sp_ref[0, T // 2:])], axis=0)
    r = lax.broadcasted_iota(jnp.int32, (T, 1), 0)
    h = (r // 16) % 2
    out_ref[0] = idx + (b * 2 + h) * Lin


def _combine_body(h_ref, w_ref, b_ref, o_ref):
    h = h_ref[0]
    h0 = h[:, 0, :]
    h1 = h[:, 1, :]
    o_ref[0] = (jnp.dot(h0, w_ref[0:128, :],
                        preferred_element_type=jnp.float32)
                + jnp.dot(h1, w_ref[128:256, :],
                          preferred_element_type=jnp.float32)
                + b_ref[...])


def _sc_gather_combine(table, idx3, attw3, n_out, d):
    rows_w = (n_out * _P) // _NW
    nch = rows_w // _CH
    out_rows_ch = _CH // _P // _P
    mesh = plsc.VectorSubcoreMesh(core_axis_name="c", subcore_axis_name="s")

    @functools.partial(
        pl.kernel,
        mesh=mesh,
        out_type=jax.ShapeDtypeStruct((n_out // _P, 4 * d), jnp.float32),
        scratch_types=[
            pltpu.VMEM((nch, _CH), jnp.int32),
            pltpu.VMEM((rows_w,), jnp.float32),
            pltpu.VMEM((_CH, 4 * d), jnp.float32),
            pltpu.VMEM((_CH, 4 * d), jnp.float32),
            pltpu.VMEM((out_rows_ch, 4 * d), jnp.float32),
            pltpu.SemaphoreType.DMA,
            pltpu.SemaphoreType.DMA,
        ],
        compiler_params=pltpu.CompilerParams(use_tc_tiling_on_sc=True,
                                             needs_layout_passes=False),
    )
    def gc_kernel(table_hbm, idx_hbm, attw_hbm, out_hbm,
                  idx_v, aw_v, raw_a, raw_b, out_v, sem_a, sem_b):
        wid = lax.axis_index("s") * _NC + lax.axis_index("c")
        pltpu.sync_copy(idx_hbm.at[wid], idx_v)
        pltpu.sync_copy(attw_hbm.at[wid], aw_v)

        def combine_chunk(c, raw_v):
            for j in range(_CH // _P):
                m = j % _M
                lb = (m % _P) * d
                pr = j // _P
                acc_lo = jnp.zeros((16,), jnp.float32)
                acc_hi = jnp.zeros((16,), jnp.float32)
                for p in range(_P):
                    n_loc = c * _CH + j * _P + p
                    w = plsc.load_gather(
                        aw_v, [jnp.full((16,), n_loc, jnp.int32)])
                    r = j * _P + p
                    acc_lo = acc_lo + w * raw_v[r, pl.ds(lb, 16)]
                    acc_hi = acc_hi + w * raw_v[r, pl.ds(lb + 16, 16)]
                out_v[pr, pl.ds(lb, 16)] = acc_lo
                out_v[pr, pl.ds(lb + 16, 16)] = acc_hi
            pltpu.sync_copy(
                out_v,
                out_hbm.at[pl.ds(wid * (nch * out_rows_ch) + c * out_rows_ch,
                                 out_rows_ch)])

        pltpu.async_copy(table_hbm.at[idx_v.at[0]], raw_a, sem_a)

        def pair(i, carry):
            ca = 2 * i
            pltpu.make_async_copy(
                table_hbm.at[idx_v.at[ca]], raw_a, sem_a).wait()
            cp_b = pltpu.async_copy(
                table_hbm.at[idx_v.at[ca + 1]], raw_b, sem_b)
            combine_chunk(ca, raw_a)
            cp_b.wait()

            @pl.when(i < nch // 2 - 1)
            def _():
                pltpu.async_copy(
                    table_hbm.at[idx_v.at[ca + 2]], raw_a, sem_a)

            combine_chunk(ca + 1, raw_b)
            return carry

        lax.fori_loop(0, nch // 2, pair, 0)

    return gc_kernel(table, idx3, attw3)


def kernel(query, query_points, input, input_points, W_off, b_off,
           W_attn, b_attn, W_val, b_val, W_out, b_out):
    B, Lq, C = query.shape
    Lin = input.shape[1]
    M, P = _M, _P
    G = M * P
    D = C // M
    N = Lq * G
    T = 1024

    f32 = jnp.float32

    W_attn_r = W_attn.reshape(C, M, P).transpose(0, 2, 1).reshape(C, G)
    b_attn_r = b_attn.reshape(M, P).transpose(1, 0).reshape(G)
    W_all = jnp.concatenate([W_off, W_attn_r], axis=1)
    b_all = jnp.concatenate([b_off, b_attn_r])[None]
    ip_t = input_points.transpose(0, 2, 1)
    ip2 = jnp.sum(input_points * input_points, axis=-1)[:, None, :]
    b_out2 = b_out[None]

    samp, attw_r, value4 = pl.pallas_call(
        _prep_body,
        grid=(B,),
        in_specs=[
            pl.BlockSpec((1, Lq, C), lambda b: (b, 0, 0)),
            pl.BlockSpec((1, Lq, 3), lambda b: (b, 0, 0)),
            pl.BlockSpec((C, 4 * G), lambda b: (0, 0)),
            pl.BlockSpec((1, 4 * G), lambda b: (0, 0)),
            pl.BlockSpec((1, Lin, C), lambda b: (b, 0, 0)),
            pl.BlockSpec((C, C), lambda b: (0, 0)),
            pl.BlockSpec((1, C), lambda b: (0, 0)),
        ],
        out_specs=[
            pl.BlockSpec((1, Lq, 3 * G), lambda b: (b, 0, 0)),
            pl.BlockSpec((1, Lq, G), lambda b: (b, 0, 0)),
            pl.BlockSpec((1, 2, Lin, 128), lambda b: (b, 0, 0, 0)),
        ],
        out_shape=[
            jax.ShapeDtypeStruct((B, Lq, 3 * G), f32),
            jax.ShapeDtypeStruct((B, Lq, G), f32),
            jax.ShapeDtypeStruct((B, 2, Lin, 128), f32),
        ],
    )(query, query_points, W_all, b_all, input, W_val, b_val[None])

    sp_flat = samp.reshape(B, N, 3)
    gidx = pl.pallas_call(
        functools.partial(_argmin_body, T=T, Lin=Lin),
        grid=(B, N // T),
        in_specs=[
            pl.BlockSpec((1, T, 3), lambda b, t: (b, t, 0)),
            pl.BlockSpec((1, 3, Lin), lambda b, t: (b, 0, 0)),
            pl.BlockSpec((1, 1, Lin), lambda b, t: (b, 0, 0)),
        ],
        out_specs=pl.BlockSpec((1, T, 1), lambda b, t: (b, t, 0)),
        out_shape=jax.ShapeDtypeStruct((B, N, 1), jnp.int32),
    )(sp_flat, ip_t, ip2)

    value_t = value4.reshape(B * 2 * Lin, 128)
    n_rows = B * N
    idx3 = gidx.reshape(_NW, n_rows // _NW // _CH, _CH)
    attw_m = attw_r.reshape(B, Lq, P, M).transpose(0, 1, 3, 2)
    attw3 = attw_m.reshape(_NW, n_rows // _NW)
    heads = _sc_gather_combine(value_t, idx3, attw3, B * Lq * M, D)

    h4 = heads.reshape(B, Lq, 2, 128)
    out = pl.pallas_call(
        _combine_body,
        grid=(B,),
        in_specs=[
            pl.BlockSpec((1, Lq, 2, 128), lambda b: (b, 0, 0, 0)),
            pl.BlockSpec((C, C), lambda b: (0, 0)),
            pl.BlockSpec((1, C), lambda b: (0, 0)),
        ],
        out_specs=pl.BlockSpec((1, Lq, C), lambda b: (b, 0, 0)),
        out_shape=jax.ShapeDtypeStruct((B, Lq, C), f32),
    )(h4, W_out, b_out2)
    return out


# speedup vs baseline: 1.0869x; 1.0187x over previous
import functools

import jax
import jax.numpy as jnp
from jax import lax
from jax.experimental import pallas as pl
from jax.experimental.pallas import tpu as pltpu
from jax.experimental.pallas import tpu_sc as plsc

_M = 8
_P = 4

_NC = 2
_NS = 16
_NW = _NC * _NS
_CH = 128


def _prep_body(q_ref, qp_ref, wall_ref, ball_ref, inp_ref, wval_ref,
               bval_ref, samp_ref, attw_ref, val_ref):
    G = _M * _P
    proj = jnp.dot(q_ref[0], wall_ref[...],
                   preferred_element_type=jnp.float32) + ball_ref[...]
    qp_t = jnp.concatenate([qp_ref[0]] * G, axis=1)
    samp_ref[0] = qp_t + proj[:, :3 * G]
    logits = proj[:, 3 * G:3 * G + G]
    sl = [logits[:, i * _M:(i + 1) * _M] for i in range(_P)]
    mx = jnp.maximum(jnp.maximum(sl[0], sl[1]), jnp.maximum(sl[2], sl[3]))
    e = jnp.exp(logits - jnp.concatenate([mx] * _P, axis=1))
    es = [e[:, i * _M:(i + 1) * _M] for i in range(_P)]
    ssum = (es[0] + es[1]) + (es[2] + es[3])
    attw_ref[0] = e / jnp.concatenate([ssum] * _P, axis=1)
    value = jnp.dot(inp_ref[0], wval_ref[...],
                    preferred_element_type=jnp.float32) + bval_ref[...]
    val_ref[0, 0] = value[:, :128]
    val_ref[0, 1] = value[:, 128:]


def _argmin_body(sp_ref, ipt_ref, ip2_ref, out_ref, *, T, Lin):
    b = pl.program_id(0)
    ipt = ipt_ref[0]
    ip2 = ip2_ref[0]

    def half(sp):
        sp2 = jnp.sum(sp * sp, axis=1, keepdims=True)
        d2 = (sp2 + ip2) - 2.0 * jnp.dot(
            sp, ipt, preferred_element_type=jnp.float32)
        mn = jnp.min(d2, axis=1, keepdims=True)
        lane = lax.broadcasted_iota(jnp.int32, d2.shape, 1)
        return jnp.min(jnp.where(d2 == mn, lane, Lin), axis=1, keepdims=True)

    idx = jnp.concatenate(
        [half(sp_ref[0, :T // 2]), half(sp_ref[0, T // 2:])], axis=0)
    r = lax.broadcasted_iota(jnp.int32, (T, 1), 0)
    h = (r // 16) % 2
    out_ref[0] = idx + (b * 2 + h) * Lin


def _combine_body(h_ref, w_ref, b_ref, o_ref):
    h = h_ref[0]
    h0 = h[:, 0, :]
    h1 = h[:, 1, :]
    o_ref[0] = (jnp.dot(h0, w_ref[0:128, :],
                        preferred_element_type=jnp.float32)
                + jnp.dot(h1, w_ref[128:256, :],
                          preferred_element_type=jnp.float32)
                + b_ref[...])


def _sc_gather_combine(table, idx3, attw3, n_out, d):
    rows_w = (n_out * _P) // _NW
    nch = rows_w // _CH
    out_rows_ch = _CH // _P // _P
    mesh = plsc.VectorSubcoreMesh(core_axis_name="c", subcore_axis_name="s")

    @functools.partial(
        pl.kernel,
        mesh=mesh,
        out_type=jax.ShapeDtypeStruct((n_out // _P, 4 * d), jnp.float32),
        scratch_types=[
            pltpu.VMEM((nch, _CH), jnp.int32),
            pltpu.VMEM((rows_w,), jnp.float32),
            pltpu.VMEM((_CH, 4 * d), jnp.float32),
            pltpu.VMEM((_CH, 4 * d), jnp.float32),
            pltpu.VMEM((out_rows_ch, 4 * d), jnp.float32),
            pltpu.SemaphoreType.DMA,
            pltpu.SemaphoreType.DMA,
        ],
        compiler_params=pltpu.CompilerParams(use_tc_tiling_on_sc=True,
                                             needs_layout_passes=False),
    )
    def gc_kernel(table_hbm, idx_hbm, attw_hbm, out_hbm,
                  idx_v, aw_v, raw_a, raw_b, out_v, sem_a, sem_b):
        wid = lax.axis_index("s") * _NC + lax.axis_index("c")
        pltpu.sync_copy(idx_hbm.at[wid], idx_v)
        pltpu.sync_copy(attw_hbm.at[wid], aw_v)

        def combine_chunk(c, raw_v):
            for j in range(_CH // _P):
                m = j % _M
                lb = (m % _P) * d
                pr = j // _P
                acc_lo = jnp.zeros((16,), jnp.float32)
                acc_hi = jnp.zeros((16,), jnp.float32)
                for p in range(_P):
                    n_loc = c * _CH + j * _P + p
                    w = plsc.load_gather(
                        aw_v, [jnp.full((16,), n_loc, jnp.int32)])
                    r = j * _P + p
                    acc_lo = acc_lo + w * raw_v[r, pl.ds(lb, 16)]
                    acc_hi = acc_hi + w * raw_v[r, pl.ds(lb + 16, 16)]
                out_v[pr, pl.ds(lb, 16)] = acc_lo
                out_v[pr, pl.ds(lb + 16, 16)] = acc_hi
            pltpu.sync_copy(
                out_v,
                out_hbm.at[pl.ds(wid * (nch * out_rows_ch) + c * out_rows_ch,
                                 out_rows_ch)])

        pltpu.async_copy(table_hbm.at[idx_v.at[0]], raw_a, sem_a)

        def pair(i, carry):
            ca = 2 * i
            pltpu.make_async_copy(
                table_hbm.at[idx_v.at[ca]], raw_a, sem_a).wait()
            cp_b = pltpu.async_copy(
                table_hbm.at[idx_v.at[ca + 1]], raw_b, sem_b)
            combine_chunk(ca, raw_a)
            cp_b.wait()

            @pl.when(i < nch // 2 - 1)
            def _():
                pltpu.async_copy(
                    table_hbm.at[idx_v.at[ca + 2]], raw_a, sem_a)

            combine_chunk(ca + 1, raw_b)
            return carry

        lax.fori_loop(0, nch // 2, pair, 0)

    return gc_kernel(table, idx3, attw3)


def kernel(query, query_points, input, input_points, W_off, b_off,
           W_attn, b_attn, W_val, b_val, W_out, b_out):
    B, Lq, C = query.shape
    Lin = input.shape[1]
    M, P = _M, _P
    G = M * P
    D = C // M
    N = Lq * G
    T = 2048

    f32 = jnp.float32

    W_attn_r = W_attn.reshape(C, M, P).transpose(0, 2, 1).reshape(C, G)
    b_attn_r = b_attn.reshape(M, P).transpose(1, 0).reshape(G)
    W_all = jnp.concatenate([W_off, W_attn_r], axis=1)
    b_all = jnp.concatenate([b_off, b_attn_r])[None]
    ip_t = input_points.transpose(0, 2, 1)
    ip2 = jnp.sum(input_points * input_points, axis=-1)[:, None, :]
    b_out2 = b_out[None]

    samp, attw_r, value4 = pl.pallas_call(
        _prep_body,
        grid=(B,),
        in_specs=[
            pl.BlockSpec((1, Lq, C), lambda b: (b, 0, 0)),
            pl.BlockSpec((1, Lq, 3), lambda b: (b, 0, 0)),
            pl.BlockSpec((C, 4 * G), lambda b: (0, 0)),
            pl.BlockSpec((1, 4 * G), lambda b: (0, 0)),
            pl.BlockSpec((1, Lin, C), lambda b: (b, 0, 0)),
            pl.BlockSpec((C, C), lambda b: (0, 0)),
            pl.BlockSpec((1, C), lambda b: (0, 0)),
        ],
        out_specs=[
            pl.BlockSpec((1, Lq, 3 * G), lambda b: (b, 0, 0)),
            pl.BlockSpec((1, Lq, G), lambda b: (b, 0, 0)),
            pl.BlockSpec((1, 2, Lin, 128), lambda b: (b, 0, 0, 0)),
        ],
        out_shape=[
            jax.ShapeDtypeStruct((B, Lq, 3 * G), f32),
            jax.ShapeDtypeStruct((B, Lq, G), f32),
            jax.ShapeDtypeStruct((B, 2, Lin, 128), f32),
        ],
    )(query, query_points, W_all, b_all, input, W_val, b_val[None])

    sp_flat = samp.reshape(B, N, 3)
    gidx = pl.pallas_call(
        functools.partial(_argmin_body, T=T, Lin=Lin),
        grid=(B, N // T),
        in_specs=[
            pl.BlockSpec((1, T, 3), lambda b, t: (b, t, 0)),
            pl.BlockSpec((1, 3, Lin), lambda b, t: (b, 0, 0)),
            pl.BlockSpec((1, 1, Lin), lambda b, t: (b, 0, 0)),
        ],
        out_specs=pl.BlockSpec((1, T, 1), lambda b, t: (b, t, 0)),
        out_shape=jax.ShapeDtypeStruct((B, N, 1), jnp.int32),
    )(sp_flat, ip_t, ip2)

    value_t = value4.reshape(B * 2 * Lin, 128)
    n_rows = B * N
    idx3 = gidx.reshape(_NW, n_rows // _NW // _CH, _CH)
    attw_m = attw_r.reshape(B, Lq, P, M).transpose(0, 1, 3, 2)
    attw3 = attw_m.reshape(_NW, n_rows // _NW)
    heads = _sc_gather_combine(value_t, idx3, attw3, B * Lq * M, D)

    h4 = heads.reshape(B, Lq, 2, 128)
    out = pl.pallas_call(
        _combine_body,
        grid=(B,),
        in_specs=[
            pl.BlockSpec((1, Lq, 2, 128), lambda b: (b, 0, 0, 0)),
            pl.BlockSpec((C, C), lambda b: (0, 0)),
            pl.BlockSpec((1, C), lambda b: (0, 0)),
        ],
        out_specs=pl.BlockSpec((1, Lq, C), lambda b: (b, 0, 0)),
        out_shape=jax.ShapeDtypeStruct((B, Lq, C), f32),
    )(h4, W_out, b_out2)
    return out


# speedup vs baseline: 1.0916x; 1.0043x over previous
import functools

import jax
import jax.numpy as jnp
from jax import lax
from jax.experimental import pallas as pl
from jax.experimental.pallas import tpu as pltpu
from jax.experimental.pallas import tpu_sc as plsc

_M = 8
_P = 4

_NC = 2
_NS = 16
_NW = _NC * _NS
_CH = 128


def _prep_body(q_ref, qp_ref, wall_ref, ball_ref, inp_ref, wval_ref,
               bval_ref, samp_ref, attw_ref, val_ref):
    G = _M * _P
    proj = jnp.dot(q_ref[0], wall_ref[...],
                   preferred_element_type=jnp.float32) + ball_ref[...]
    qp_t = jnp.concatenate([qp_ref[0]] * G, axis=1)
    samp_ref[0] = qp_t + proj[:, :3 * G]
    logits = proj[:, 3 * G:3 * G + G]
    sl = [logits[:, i * _M:(i + 1) * _M] for i in range(_P)]
    mx = jnp.maximum(jnp.maximum(sl[0], sl[1]), jnp.maximum(sl[2], sl[3]))
    e = jnp.exp(logits - jnp.concatenate([mx] * _P, axis=1))
    es = [e[:, i * _M:(i + 1) * _M] for i in range(_P)]
    ssum = (es[0] + es[1]) + (es[2] + es[3])
    attw_ref[0] = e / jnp.concatenate([ssum] * _P, axis=1)
    value = jnp.dot(inp_ref[0], wval_ref[...],
                    preferred_element_type=jnp.float32) + bval_ref[...]
    val_ref[0, 0] = value[:, :128]
    val_ref[0, 1] = value[:, 128:]


def _argmin_body(sp_ref, ipt_ref, ip2_ref, out_ref, *, T, Lin):
    b = pl.program_id(0)
    ipt = ipt_ref[0]
    ip2 = ip2_ref[0]

    def half(sp):
        sp2 = jnp.sum(sp * sp, axis=1, keepdims=True)
        d2 = (sp2 + ip2) - 2.0 * jnp.dot(
            sp, ipt, preferred_element_type=jnp.float32)
        mn = jnp.min(d2, axis=1, keepdims=True)
        lane = lax.broadcasted_iota(jnp.int32, d2.shape, 1)
        return jnp.min(jnp.where(d2 == mn, lane, Lin), axis=1, keepdims=True)

    idx = jnp.concatenate(
        [half(sp_ref[0, :T // 2]), half(sp_ref[0, T // 2:])], axis=0)
    r = lax.broadcasted_iota(jnp.int32, (T, 1), 0)
    h = (r // 16) % 2
    out_ref[0] = idx + (b * 2 + h) * Lin


def _combine_body(h_ref, w_ref, b_ref, o_ref):
    h = h_ref[0]
    h0 = h[:, 0, :]
    h1 = h[:, 1, :]
    o_ref[0] = (jnp.dot(h0, w_ref[0:128, :],
                        preferred_element_type=jnp.float32)
                + jnp.dot(h1, w_ref[128:256, :],
                          preferred_element_type=jnp.float32)
                + b_ref[...])


def _sc_gather_combine(table, idx3, attw3, n_out, d):
    rows_w = (n_out * _P) // _NW
    nch = rows_w // _CH
    out_rows_ch = _CH // _P // _P
    mesh = plsc.VectorSubcoreMesh(core_axis_name="c", subcore_axis_name="s")

    @functools.partial(
        pl.kernel,
        mesh=mesh,
        out_type=jax.ShapeDtypeStruct((n_out // _P, 4 * d), jnp.float32),
        scratch_types=[
            pltpu.VMEM((nch, _CH), jnp.int32),
            pltpu.VMEM((rows_w,), jnp.float32),
            pltpu.VMEM((_CH, 4 * d), jnp.float32),
            pltpu.VMEM((_CH, 4 * d), jnp.float32),
            pltpu.VMEM((out_rows_ch, 4 * d), jnp.float32),
            pltpu.SemaphoreType.DMA,
            pltpu.SemaphoreType.DMA,
        ],
        compiler_params=pltpu.CompilerParams(use_tc_tiling_on_sc=True,
                                             needs_layout_passes=False),
    )
    def gc_kernel(table_hbm, idx_hbm, attw_hbm, out_hbm,
                  idx_v, aw_v, raw_a, raw_b, out_v, sem_a, sem_b):
        wid = lax.axis_index("s") * _NC + lax.axis_index("c")
        pltpu.sync_copy(idx_hbm.at[wid], idx_v)
        pltpu.sync_copy(attw_hbm.at[wid], aw_v)

        def combine_chunk(c, raw_v):
            for j in range(_CH // _P):
                m = j % _M
                lb = (m % _P) * d
                pr = j // _P
                acc_lo = jnp.zeros((16,), jnp.float32)
                acc_hi = jnp.zeros((16,), jnp.float32)
                for p in range(_P):
                    n_loc = c * _CH + j * _P + p
                    w = plsc.load_gather(
                        aw_v, [jnp.full((16,), n_loc, jnp.int32)])
                    r = j * _P + p
                    acc_lo = acc_lo + w * raw_v[r, pl.ds(lb, 16)]
                    acc_hi = acc_hi + w * raw_v[r, pl.ds(lb + 16, 16)]
                out_v[pr, pl.ds(lb, 16)] = acc_lo
                out_v[pr, pl.ds(lb + 16, 16)] = acc_hi
            pltpu.sync_copy(
                out_v,
                out_hbm.at[pl.ds(wid * (nch * out_rows_ch) + c * out_rows_ch,
                                 out_rows_ch)])

        pltpu.async_copy(table_hbm.at[idx_v.at[0]], raw_a, sem_a)

        def pair(i, carry):
            ca = 2 * i
            pltpu.make_async_copy(
                table_hbm.at[idx_v.at[ca]], raw_a, sem_a).wait()
            cp_b = pltpu.async_copy(
                table_hbm.at[idx_v.at[ca + 1]], raw_b, sem_b)
            combine_chunk(ca, raw_a)
            cp_b.wait()

            @pl.when(i < nch // 2 - 1)
            def _():
                pltpu.async_copy(
                    table_hbm.at[idx_v.at[ca + 2]], raw_a, sem_a)

            combine_chunk(ca + 1, raw_b)
            return carry

        lax.fori_loop(0, nch // 2, pair, 0)

    return gc_kernel(table, idx3, attw3)


def kernel(query, query_points, input, input_points, W_off, b_off,
           W_attn, b_attn, W_val, b_val, W_out, b_out):
    B, Lq, C = query.shape
    Lin = input.shape[1]
    M, P = _M, _P
    G = M * P
    D = C // M
    N = Lq * G
    T = 4096

    f32 = jnp.float32

    W_attn_r = W_attn.reshape(C, M, P).transpose(0, 2, 1).reshape(C, G)
    b_attn_r = b_attn.reshape(M, P).transpose(1, 0).reshape(G)
    W_all = jnp.concatenate([W_off, W_attn_r], axis=1)
    b_all = jnp.concatenate([b_off, b_attn_r])[None]
    ip_t = input_points.transpose(0, 2, 1)
    ip2 = jnp.sum(input_points * input_points, axis=-1)[:, None, :]
    b_out2 = b_out[None]

    samp, attw_r, value4 = pl.pallas_call(
        _prep_body,
        grid=(B,),
        in_specs=[
            pl.BlockSpec((1, Lq, C), lambda b: (b, 0, 0)),
            pl.BlockSpec((1, Lq, 3), lambda b: (b, 0, 0)),
            pl.BlockSpec((C, 4 * G), lambda b: (0, 0)),
            pl.BlockSpec((1, 4 * G), lambda b: (0, 0)),
            pl.BlockSpec((1, Lin, C), lambda b: (b, 0, 0)),
            pl.BlockSpec((C, C), lambda b: (0, 0)),
            pl.BlockSpec((1, C), lambda b: (0, 0)),
        ],
        out_specs=[
            pl.BlockSpec((1, Lq, 3 * G), lambda b: (b, 0, 0)),
            pl.BlockSpec((1, Lq, G), lambda b: (b, 0, 0)),
            pl.BlockSpec((1, 2, Lin, 128), lambda b: (b, 0, 0, 0)),
        ],
        out_shape=[
            jax.ShapeDtypeStruct((B, Lq, 3 * G), f32),
            jax.ShapeDtypeStruct((B, Lq, G), f32),
            jax.ShapeDtypeStruct((B, 2, Lin, 128), f32),
        ],
    )(query, query_points, W_all, b_all, input, W_val, b_val[None])

    sp_flat = samp.reshape(B, N, 3)
    gidx = pl.pallas_call(
        functools.partial(_argmin_body, T=T, Lin=Lin),
        grid=(B, N // T),
        in_specs=[
            pl.BlockSpec((1, T, 3), lambda b, t: (b, t, 0)),
            pl.BlockSpec((1, 3, Lin), lambda b, t: (b, 0, 0)),
            pl.BlockSpec((1, 1, Lin), lambda b, t: (b, 0, 0)),
        ],
        out_specs=pl.BlockSpec((1, T, 1), lambda b, t: (b, t, 0)),
        out_shape=jax.ShapeDtypeStruct((B, N, 1), jnp.int32),
    )(sp_flat, ip_t, ip2)

    value_t = value4.reshape(B * 2 * Lin, 128)
    n_rows = B * N
    idx3 = gidx.reshape(_NW, n_rows // _NW // _CH, _CH)
    attw_m = attw_r.reshape(B, Lq, P, M).transpose(0, 1, 3, 2)
    attw3 = attw_m.reshape(_NW, n_rows // _NW)
    heads = _sc_gather_combine(value_t, idx3, attw3, B * Lq * M, D)

    h4 = heads.reshape(B, Lq, 2, 128)
    out = pl.pallas_call(
        _combine_body,
        grid=(B,),
        in_specs=[
            pl.BlockSpec((1, Lq, 2, 128), lambda b: (b, 0, 0, 0)),
            pl.BlockSpec((C, C), lambda b: (0, 0)),
            pl.BlockSpec((1, C), lambda b: (0, 0)),
        ],
        out_specs=pl.BlockSpec((1, Lq, C), lambda b: (b, 0, 0)),
        out_shape=jax.ShapeDtypeStruct((B, Lq, C), f32),
    )(h4, W_out, b_out2)
    return out
